# baseline (speedup 1.0000x reference)
import jax
import jax.numpy as jnp
from jax.experimental import pallas as pl

B, C, H, W = 128, 96, 32, 32
HW = H * W
BB = 16


def _tc_kernel(x_ref, row_ref, col_ref, out_ref):
    row = row_ref[:]
    col = col_ref[:]
    pos = (row[:, None, :] + col[None, :, :]).reshape(HW, C)
    for i in range(BB):
        out_ref[i] = jnp.full((HW * C // 128, 128), x_ref[i][0, 0]) + pos[0, 0]


def kernel(x, row_embed, col_embed):
    x3 = x.reshape(B, C, HW)
    out = pl.pallas_call(
        _tc_kernel,
        grid=(B // BB,),
        in_specs=[
            pl.BlockSpec((BB, C, HW), lambda b: (b, 0, 0)),
            pl.BlockSpec((H, C), lambda b: (0, 0)),
            pl.BlockSpec((W, C), lambda b: (0, 0)),
        ],
        out_specs=pl.BlockSpec((BB, HW * C // 128, 128), lambda b: (b, 0, 0)),
        out_shape=jax.ShapeDtypeStruct((B, HW * C // 128, 128), jnp.float32),
    )(x3, row_embed, col_embed)
    return out.reshape(B, HW, C)


# speedup vs baseline: 4.3619x; 4.3619x over previous
import jax
import jax.numpy as jnp
from jax.experimental import pallas as pl

B, C, H, W = 128, 96, 32, 32
HW = H * W
CB = 8


def _tc_kernel(x_ref, rowt_ref, colt_ref, out_ref):
    p_i = jax.lax.broadcasted_iota(jnp.int32, (H, HW), 1)
    s_i = jax.lax.broadcasted_iota(jnp.int32, (H, HW), 0)
    sel_h = (p_i // W == s_i).astype(jnp.float32)
    sel_w = (p_i % W == s_i).astype(jnp.float32)
    posc = jax.lax.dot_general(
        rowt_ref[:], sel_h, (((1,), (0,)), ((), ())),
        preferred_element_type=jnp.float32,
        precision=jax.lax.Precision.HIGHEST,
    ) + jax.lax.dot_general(
        colt_ref[:], sel_w, (((1,), (0,)), ((), ())),
        preferred_element_type=jnp.float32,
        precision=jax.lax.Precision.HIGHEST,
    )
    xs = x_ref[:].reshape(CB, HW, B)
    tt = jnp.transpose(xs, (0, 2, 1))
    out_ref[:] = jnp.transpose(tt, (1, 0, 2)) + posc[None, :, :]


def kernel(x, row_embed, col_embed):
    xv = jnp.transpose(x, (1, 2, 3, 0))
    pout = pl.pallas_call(
        _tc_kernel,
        grid=(C // CB,),
        in_specs=[
            pl.BlockSpec((CB, H, W, B), lambda c: (c, 0, 0, 0)),
            pl.BlockSpec((CB, H), lambda c: (c, 0)),
            pl.BlockSpec((CB, W), lambda c: (c, 0)),
        ],
        out_specs=pl.BlockSpec((B, CB, HW), lambda c: (0, c, 0)),
        out_shape=jax.ShapeDtypeStruct((B, C, HW), jnp.float32),
    )(xv, row_embed.T, col_embed.T)
    return jnp.transpose(pout, (0, 2, 1))


# speedup vs baseline: 4.4856x; 1.0284x over previous
import jax
import jax.numpy as jnp
from jax.experimental import pallas as pl

B, C, H, W = 128, 96, 32, 32
HW = H * W
CB = 16


def _tc_kernel(x_ref, rowt_ref, colt_ref, out_ref):
    p_i = jax.lax.broadcasted_iota(jnp.int32, (H, HW), 1)
    s_i = jax.lax.broadcasted_iota(jnp.int32, (H, HW), 0)
    sel_h = (p_i // W == s_i).astype(jnp.float32)
    sel_w = (p_i % W == s_i).astype(jnp.float32)
    posc = jax.lax.dot_general(
        rowt_ref[:], sel_h, (((1,), (0,)), ((), ())),
        preferred_element_type=jnp.float32,
        precision=jax.lax.Precision.HIGHEST,
    ) + jax.lax.dot_general(
        colt_ref[:], sel_w, (((1,), (0,)), ((), ())),
        preferred_element_type=jnp.float32,
        precision=jax.lax.Precision.HIGHEST,
    )
    xs = x_ref[:].reshape(CB, HW, B)
    tt = jnp.transpose(xs, (0, 2, 1))
    out_ref[:] = jnp.transpose(tt, (1, 0, 2)) + posc[None, :, :]


def kernel(x, row_embed, col_embed):
    xv = jnp.transpose(x, (1, 2, 3, 0))
    pout = pl.pallas_call(
        _tc_kernel,
        grid=(C // CB,),
        in_specs=[
            pl.BlockSpec((CB, H, W, B), lambda c: (c, 0, 0, 0)),
            pl.BlockSpec((CB, H), lambda c: (c, 0)),
            pl.BlockSpec((CB, W), lambda c: (c, 0)),
        ],
        out_specs=pl.BlockSpec((B, CB, HW), lambda c: (0, c, 0)),
        out_shape=jax.ShapeDtypeStruct((B, C, HW), jnp.float32),
    )(xv, row_embed.T, col_embed.T)
    return jnp.transpose(pout, (0, 2, 1))
